# baseline (speedup 1.0000x reference)
import functools

import jax
import jax.numpy as jnp
from jax.experimental import pallas as pl

_DEPTH = 4
_K = 1024
_D = 64


def _level(r, cb, cb_norms, cb_cat, iota):
    r_norms = jnp.sum(r * r, axis=1, keepdims=True)
    cross2 = jax.lax.dot_general(
        r * -2.0, cb,
        dimension_numbers=(((1,), (1,)), ((), ())),
        preferred_element_type=jnp.float32,
    )
    d = (r_norms + cross2) + cb_norms[None, :]
    idx = jnp.argmin(d, axis=1).astype(jnp.int32)
    onehot = (iota == idx[:, None]).astype(jnp.bfloat16)
    q3 = jax.lax.dot_general(onehot, cb_cat,
                             dimension_numbers=(((1,), (0,)), ((), ())),
                             preferred_element_type=jnp.float32)
    q = (q3[:, :_D] + q3[:, _D:2 * _D]) + q3[:, 2 * _D:]
    return q, idx


def _rvq_block(lat_ref, cb_ref, out_ref, idx_ref, *, bt):
    h = bt // 2
    iota = jax.lax.broadcasted_iota(jnp.int32, (h, _K), 1)
    cb_all = cb_ref[...]
    cb_hi = cb_all.astype(jnp.bfloat16)
    res1 = cb_all - cb_hi.astype(jnp.float32)
    cb_mid = res1.astype(jnp.bfloat16)
    res2 = res1 - cb_mid.astype(jnp.float32)
    cb_lo = res2.astype(jnp.bfloat16)
    cb_cat = jnp.concatenate([cb_hi, cb_mid, cb_lo], axis=-1)
    cb_norms = [jnp.sum(cb_all[l] * cb_all[l], axis=1) for l in range(_DEPTH)]

    lat_a = lat_ref[:h, :]
    lat_b = lat_ref[h:, :]
    ra, rb = lat_a, lat_b
    qsum_a = jnp.zeros_like(lat_a)
    qsum_b = jnp.zeros_like(lat_b)
    for level in range(_DEPTH):
        qa, idx_a = _level(ra, cb_all[level], cb_norms[level],
                           cb_cat[level], iota)
        qb, idx_b = _level(rb, cb_all[level], cb_norms[level],
                           cb_cat[level], iota)
        qsum_a = qsum_a + qa
        ra = ra - qa
        idx_ref[0, level, :h] = idx_a
        qsum_b = qsum_b + qb
        rb = rb - qb
        idx_ref[0, level, h:] = idx_b
    out_ref[:h, :] = lat_a + (qsum_a - lat_a)
    out_ref[h:, :] = lat_b + (qsum_b - lat_b)


def _rvq(latent, codebooks, bt):
    n, d = latent.shape
    nb = n // bt
    out, idx = pl.pallas_call(
        functools.partial(_rvq_block, bt=bt),
        grid=(nb,),
        in_specs=[
            pl.BlockSpec((bt, d), lambda i: (i, 0)),
            pl.BlockSpec((_DEPTH, _K, _D), lambda i: (0, 0, 0)),
        ],
        out_specs=[
            pl.BlockSpec((bt, d), lambda i: (i, 0)),
            pl.BlockSpec((1, _DEPTH, bt), lambda i: (i, 0, 0)),
        ],
        out_shape=[
            jax.ShapeDtypeStruct((n, d), jnp.float32),
            jax.ShapeDtypeStruct((nb, _DEPTH, bt), jnp.int32),
        ],
    )(latent, codebooks)
    return out, idx


def kernel(latent, codebooks):
    bt = 2048
    out, idx = _rvq(latent, codebooks, bt)
    n = latent.shape[0]
    indices = jnp.transpose(idx, (1, 0, 2)).reshape(_DEPTH, n)
    return out, indices


# speedup vs baseline: 1.2598x; 1.2598x over previous
import functools

import jax
import jax.numpy as jnp
from jax.experimental import pallas as pl

_DEPTH = 4
_K = 1024
_D = 64


def _rvq_block(lat_ref, cb_ref, cbt_ref, cbcat_ref, out_ref, idx_ref, *, bt):
    lat = lat_ref[...]
    r = lat
    qsum = jnp.zeros_like(lat)
    iota = jax.lax.broadcasted_iota(jnp.int32, (bt, _K), 1)
    cb_all = cb_ref[...]
    cb_norms = [jnp.sum(cb_all[l] * cb_all[l], axis=1) for l in range(_DEPTH)]
    for level in range(_DEPTH):
        cross2 = jax.lax.dot_general(
            r * -2.0, cbt_ref[level],
            dimension_numbers=(((1,), (0,)), ((), ())),
            preferred_element_type=jnp.float32,
        )
        r_norms = jnp.sum(r * r, axis=1, keepdims=True)
        d = (r_norms + cross2) + cb_norms[level][None, :]
        idx = jnp.argmin(d, axis=1).astype(jnp.int32)
        onehot = (iota == idx[:, None]).astype(jnp.bfloat16)
        q3 = jax.lax.dot_general(onehot, cbcat_ref[level],
                                 dimension_numbers=(((1,), (0,)), ((), ())),
                                 preferred_element_type=jnp.float32)
        q = (q3[:, :_D] + q3[:, _D:2 * _D]) + q3[:, 2 * _D:]
        qsum = qsum + q
        r = r - q
        idx_ref[0, level, :] = idx
    out_ref[...] = lat + (qsum - lat)


def _rvq(latent, codebooks, cbt, cb_cat, bt):
    n, d = latent.shape
    nb = n // bt
    out, idx = pl.pallas_call(
        functools.partial(_rvq_block, bt=bt),
        grid=(nb,),
        in_specs=[
            pl.BlockSpec((bt, d), lambda i: (i, 0)),
            pl.BlockSpec((_DEPTH, _K, _D), lambda i: (0, 0, 0)),
            pl.BlockSpec((_DEPTH, _D, _K), lambda i: (0, 0, 0)),
            pl.BlockSpec((_DEPTH, _K, 3 * _D), lambda i: (0, 0, 0)),
        ],
        out_specs=[
            pl.BlockSpec((bt, d), lambda i: (i, 0)),
            pl.BlockSpec((1, _DEPTH, bt), lambda i: (i, 0, 0)),
        ],
        out_shape=[
            jax.ShapeDtypeStruct((n, d), jnp.float32),
            jax.ShapeDtypeStruct((nb, _DEPTH, bt), jnp.int32),
        ],
    )(latent, codebooks, cbt, cb_cat)
    return out, idx


def kernel(latent, codebooks):
    bt = 2048
    cb_hi = codebooks.astype(jnp.bfloat16)
    res1 = codebooks - cb_hi.astype(jnp.float32)
    cb_mid = res1.astype(jnp.bfloat16)
    res2 = res1 - cb_mid.astype(jnp.float32)
    cb_lo = res2.astype(jnp.bfloat16)
    cb_cat = jnp.concatenate([cb_hi, cb_mid, cb_lo], axis=-1)
    cbt = jnp.transpose(codebooks, (0, 2, 1))
    out, idx = _rvq(latent, codebooks, cbt, cb_cat, bt)
    n = latent.shape[0]
    indices = jnp.transpose(idx, (1, 0, 2)).reshape(_DEPTH, n)
    return out, indices


# speedup vs baseline: 1.2777x; 1.0142x over previous
import functools

import jax
import jax.numpy as jnp
from jax.experimental import pallas as pl

_DEPTH = 4
_K = 1024
_D = 64


def _rvq_block(lat_ref, cbn_ref, cbt_ref, cbcat_ref, out_ref, idx_ref, *, bt):
    lat = lat_ref[...]
    r = lat
    qsum = jnp.zeros_like(lat)
    iota = jax.lax.broadcasted_iota(jnp.int32, (bt, _K), 1)
    cb_norms = [cbn_ref[l] for l in range(_DEPTH)]
    for level in range(_DEPTH):
        cross2 = jax.lax.dot_general(
            r * -2.0, cbt_ref[level],
            dimension_numbers=(((1,), (0,)), ((), ())),
            preferred_element_type=jnp.float32,
        )
        r_norms = jnp.sum(r * r, axis=1, keepdims=True)
        d = (r_norms + cross2) + cb_norms[level][None, :]
        idx = jnp.argmin(d, axis=1).astype(jnp.int32)
        onehot = (iota == idx[:, None]).astype(jnp.bfloat16)
        q3 = jax.lax.dot_general(onehot, cbcat_ref[level],
                                 dimension_numbers=(((1,), (0,)), ((), ())),
                                 preferred_element_type=jnp.float32)
        q = (q3[:, :_D] + q3[:, _D:2 * _D]) + q3[:, 2 * _D:]
        qsum = qsum + q
        r = r - q
        idx_ref[0, level, :] = idx
    out_ref[...] = lat + (qsum - lat)


def _rvq(latent, cb_norms, cbt, cb_cat, bt):
    n, d = latent.shape
    nb = n // bt
    out, idx = pl.pallas_call(
        functools.partial(_rvq_block, bt=bt),
        grid=(nb,),
        in_specs=[
            pl.BlockSpec((bt, d), lambda i: (i, 0)),
            pl.BlockSpec((_DEPTH, _K), lambda i: (0, 0)),
            pl.BlockSpec((_DEPTH, _D, _K), lambda i: (0, 0, 0)),
            pl.BlockSpec((_DEPTH, _K, 3 * _D), lambda i: (0, 0, 0)),
        ],
        out_specs=[
            pl.BlockSpec((bt, d), lambda i: (i, 0)),
            pl.BlockSpec((1, _DEPTH, bt), lambda i: (i, 0, 0)),
        ],
        out_shape=[
            jax.ShapeDtypeStruct((n, d), jnp.float32),
            jax.ShapeDtypeStruct((nb, _DEPTH, bt), jnp.int32),
        ],
    )(latent, cb_norms, cbt, cb_cat)
    return out, idx


def kernel(latent, codebooks):
    bt = 2048
    cb_hi = codebooks.astype(jnp.bfloat16)
    res1 = codebooks - cb_hi.astype(jnp.float32)
    cb_mid = res1.astype(jnp.bfloat16)
    res2 = res1 - cb_mid.astype(jnp.float32)
    cb_lo = res2.astype(jnp.bfloat16)
    cb_cat = jnp.concatenate([cb_hi, cb_mid, cb_lo], axis=-1)
    cbt = jnp.transpose(codebooks, (0, 2, 1))
    cb_norms = jnp.sum(codebooks ** 2, axis=2)
    out, idx = _rvq(latent, cb_norms, cbt, cb_cat, bt)
    n = latent.shape[0]
    indices = jnp.transpose(idx, (1, 0, 2)).reshape(_DEPTH, n)
    return out, indices
